# baseline (speedup 1.0000x reference)
import functools

import jax
import jax.numpy as jnp
from jax import lax
from jax.experimental import pallas as pl
from jax.experimental.pallas import tpu as pltpu
from jax.experimental.pallas import tpu_sc as plsc

B = 16384
MF_DIM = 16
MLP_HALF = 32
NC = 2
NS = 16
NW = NC * NS
BPW = B // NW
CH = 128
NCH = BPW // CH

_mesh = plsc.VectorSubcoreMesh(core_axis_name="c", subcore_axis_name="s")


@functools.partial(
    pl.kernel,
    mesh=_mesh,
    compiler_params=pltpu.CompilerParams(use_tc_tiling_on_sc=False),
    out_type=[
        jax.ShapeDtypeStruct((B, MLP_HALF), jnp.float32),
        jax.ShapeDtypeStruct((B, MLP_HALF), jnp.float32),
        jax.ShapeDtypeStruct((B, MF_DIM), jnp.float32),
        jax.ShapeDtypeStruct((B, MF_DIM), jnp.float32),
    ],
    scratch_types=[
        pltpu.VMEM((BPW,), jnp.int32),
        pltpu.VMEM((BPW,), jnp.int32),
        pltpu.VMEM((BPW, MLP_HALF), jnp.float32),
        pltpu.VMEM((BPW, MLP_HALF), jnp.float32),
        pltpu.VMEM((BPW, MF_DIM), jnp.float32),
        pltpu.VMEM((BPW, MF_DIM), jnp.float32),
        pltpu.SemaphoreType.DMA,
    ],
)
def _sc_gather(user_ids, item_ids, mlp_user, mlp_item, mf_user, mf_item,
               u_out, i_out, fu_out, fi_out,
               uidx, iidx, urows, irows, furows, firows, sem):
    wid = lax.axis_index("s") * NC + lax.axis_index("c")
    base = wid * BPW
    pltpu.sync_copy(user_ids.at[pl.ds(base, BPW)], uidx)
    pltpu.sync_copy(item_ids.at[pl.ds(base, BPW)], iidx)
    copies = []
    for j in range(NCH):
        sl = pl.ds(j * CH, CH)
        copies.append(pltpu.async_copy(mlp_user.at[uidx.at[sl]], urows.at[sl], sem))
        copies.append(pltpu.async_copy(mlp_item.at[iidx.at[sl]], irows.at[sl], sem))
        copies.append(pltpu.async_copy(mf_user.at[uidx.at[sl]], furows.at[sl], sem))
        copies.append(pltpu.async_copy(mf_item.at[iidx.at[sl]], firows.at[sl], sem))
    for c in copies:
        c.wait()
    pltpu.sync_copy(urows, u_out.at[pl.ds(base, BPW)])
    pltpu.sync_copy(irows, i_out.at[pl.ds(base, BPW)])
    pltpu.sync_copy(furows, fu_out.at[pl.ds(base, BPW)])
    pltpu.sync_copy(firows, fi_out.at[pl.ds(base, BPW)])


BT = 2048


def _tc_body(u_ref, i_ref, fu_ref, fi_ref, W1_ref, b1_ref, W2_ref, b2_ref,
             W3_ref, b3_ref, W4_ref, b4_ref, out_ref):
    x = jnp.concatenate([u_ref[...], i_ref[...]], axis=1)
    h = jnp.maximum(jnp.dot(x, W1_ref[...], preferred_element_type=jnp.float32)
                    + b1_ref[...], 0.0)
    h = jnp.maximum(jnp.dot(h, W2_ref[...], preferred_element_type=jnp.float32)
                    + b2_ref[...], 0.0)
    h = jnp.maximum(jnp.dot(h, W3_ref[...], preferred_element_type=jnp.float32)
                    + b3_ref[...], 0.0)
    mf = jnp.sum(fu_ref[...] * fi_ref[...], axis=1, keepdims=True)
    z = (mf * W4_ref[0:1, :]
         + jnp.dot(h, W4_ref[1:9, :], preferred_element_type=jnp.float32)
         + b4_ref[...])
    out_ref[...] = 1.0 / (1.0 + jnp.exp(-z))


def _tc_mlp(urows, irows, furows, firows, W1, b1r, W2, b2r, W3, b3r, W4p, b4r):
    grid = (B // BT,)
    full = lambda g: (0, 0)
    tile = lambda g: (g, 0)
    return pl.pallas_call(
        _tc_body,
        grid=grid,
        in_specs=[
            pl.BlockSpec((BT, MLP_HALF), tile),
            pl.BlockSpec((BT, MLP_HALF), tile),
            pl.BlockSpec((BT, MF_DIM), tile),
            pl.BlockSpec((BT, MF_DIM), tile),
            pl.BlockSpec((64, 32), full),
            pl.BlockSpec((1, 32), full),
            pl.BlockSpec((32, 16), full),
            pl.BlockSpec((1, 16), full),
            pl.BlockSpec((16, 8), full),
            pl.BlockSpec((1, 8), full),
            pl.BlockSpec((16, 1), full),
            pl.BlockSpec((1, 1), full),
        ],
        out_specs=pl.BlockSpec((BT, 1), tile),
        out_shape=jax.ShapeDtypeStruct((B, 1), jnp.float32),
    )(urows, irows, furows, firows, W1, b1r, W2, b2r, W3, b3r, W4p, b4r)


def kernel(user_ids, item_ids, mf_user, mf_item, mlp_user, mlp_item,
           W1, b1, W2, b2, W3, b3, W4, b4):
    urows, irows, furows, firows = _sc_gather(
        user_ids, item_ids, mlp_user, mlp_item, mf_user, mf_item)
    W4p = jnp.pad(W4, ((0, 7), (0, 0)))
    return _tc_mlp(urows, irows, furows, firows,
                   W1, b1.reshape(1, 32), W2, b2.reshape(1, 16),
                   W3, b3.reshape(1, 8), W4p, b4.reshape(1, 1))
